# Initial kernel scaffold; baseline (speedup 1.0000x reference)
import jax
import jax.numpy as jnp
from jax.experimental import pallas as pl


def kernel(x):
    raise NotImplementedError("write your pallas kernel here")


# speedup vs baseline: 15.5062x; 15.5062x over previous
import jax
import jax.numpy as jnp
from jax.experimental import pallas as pl


ROWS_PER_BLOCK = 512


def _maxpool_body(x_ref, o_ref):
    x = x_ref[...]
    m = jnp.max(x, axis=1, keepdims=True)
    col = jax.lax.broadcasted_iota(jnp.int32, x.shape, 1)
    idx = jnp.min(jnp.where(x == m, col, x.shape[1]), axis=1, keepdims=True)
    o_ref[...] = jnp.where(col == idx, x, 0.0)


def kernel(x):
    n_rows, n_cols = x.shape
    grid = (n_rows // ROWS_PER_BLOCK,)
    return pl.pallas_call(
        _maxpool_body,
        grid=grid,
        in_specs=[pl.BlockSpec((ROWS_PER_BLOCK, n_cols), lambda i: (i, 0))],
        out_specs=pl.BlockSpec((ROWS_PER_BLOCK, n_cols), lambda i: (i, 0)),
        out_shape=jax.ShapeDtypeStruct((n_rows, n_cols), x.dtype),
    )(x)
